# Initial kernel scaffold; baseline (speedup 1.0000x reference)
import jax
import jax.numpy as jnp
from jax.experimental import pallas as pl


def kernel(input_ids, token_emb_weight):
    raise NotImplementedError("write your pallas kernel here")


# speedup vs baseline: 1.4742x; 1.4742x over previous
import functools
import math

import jax
import jax.numpy as jnp
from jax import lax
from jax.experimental import pallas as pl
from jax.experimental.pallas import tpu as pltpu
from jax.experimental.pallas import tpu_sc as plsc

_D = 768
_SCALE = math.sqrt(float(_D))
_NC = 2
_NS = 16
_NW = _NC * _NS
_LANES = 16
_CHUNK = 64


@functools.cache
def _emb_call(n_tokens: int):
    b_per_w = n_tokens // _NW
    n_chunks = b_per_w // _CHUNK
    mesh = plsc.VectorSubcoreMesh(core_axis_name="c", subcore_axis_name="s")

    @functools.partial(
        pl.kernel,
        mesh=mesh,
        out_type=jax.ShapeDtypeStruct((n_tokens, _D), jnp.float32),
        scratch_types=[
            pltpu.VMEM((b_per_w,), jnp.int32),
            pltpu.VMEM((2, _CHUNK, _D), jnp.float32),
            pltpu.SemaphoreType.DMA,
            pltpu.SemaphoreType.DMA,
        ],
    )
    def run(idx_hbm, table_hbm, out_hbm, idx_v, buf, gsem, ssem):
        wid = lax.axis_index("s") * _NC + lax.axis_index("c")
        base = wid * b_per_w
        pltpu.sync_copy(idx_hbm.at[pl.ds(base, b_per_w)], idx_v)

        def gather(c, slot):
            return pltpu.async_copy(
                table_hbm.at[idx_v.at[pl.ds(c * _CHUNK, _CHUNK)]],
                buf.at[slot], gsem)

        def scatter(c, slot):
            return pltpu.async_copy(
                buf.at[slot],
                out_hbm.at[pl.ds(base + c * _CHUNK, _CHUNK)], ssem)

        def scale(slot):
            bref = buf.at[slot]

            def row(r, carry):
                for j in range(_D // _LANES):
                    sl = pl.ds(j * _LANES, _LANES)
                    bref[r, sl] = bref[r, sl] * _SCALE
                return carry

            lax.fori_loop(0, _CHUNK, row, 0)

        pend_g = [None, None]
        pend_s = [None, None]
        pend_g[0] = gather(0, 0)
        for c in range(n_chunks):
            slot = c % 2
            pend_g[slot].wait()
            pend_g[slot] = None
            if c + 1 < n_chunks:
                nslot = 1 - slot
                if pend_s[nslot] is not None:
                    pend_s[nslot].wait()
                    pend_s[nslot] = None
                pend_g[nslot] = gather(c + 1, nslot)
            scale(slot)
            pend_s[slot] = scatter(c, slot)
        for t in pend_s:
            if t is not None:
                t.wait()

    return run


@jax.jit
def kernel(input_ids, token_emb_weight):
    b, t = input_ids.shape
    idx = input_ids.reshape(b * t).astype(jnp.int32)
    out = _emb_call(b * t)(idx, token_emb_weight)
    return out.reshape(b, t, _D)
